# baseline (speedup 1.0000x reference)
import functools

import jax
import jax.numpy as jnp
from jax import lax
from jax.experimental import pallas as pl
from jax.experimental.pallas import tpu as pltpu
from jax.experimental.pallas import tpu_sc as plsc

B, S, D = 4, 8192, 768
NC, NS = 2, 16
NW = NC * NS
S_PER_W = S // NW
CHUNK = 64
STEPS = S_PER_W // CHUNK
CW = CHUNK * D
LANES = 16
VECS_PER_CHUNK = CW // LANES
UNROLL = 8


def _body(x_hbm, pos_hbm, out_hbm, p_v, x_v):
    wid = lax.axis_index("s") * NC + lax.axis_index("c")
    s0 = wid * S_PER_W
    for t in range(STEPS):
        pos_off = (s0 + t * CHUNK) * D
        pltpu.sync_copy(pos_hbm.at[pl.ds(pos_off, CW)], p_v)
        for b in range(B):
            x_off = b * (S * D) + pos_off
            pltpu.sync_copy(x_hbm.at[pl.ds(x_off, CW)], x_v)

            def add_body(i, _, xv=x_v, pv=p_v):
                base = i * (LANES * UNROLL)
                for u in range(UNROLL):
                    o = base + u * LANES
                    xv[pl.ds(o, LANES)] = xv[pl.ds(o, LANES)] + pv[pl.ds(o, LANES)]
                return 0

            lax.fori_loop(0, VECS_PER_CHUNK // UNROLL, add_body, 0)
            pltpu.sync_copy(x_v, out_hbm.at[pl.ds(x_off, CW)])


@jax.jit
def _pos_add(x_flat, pos_flat):
    mesh = plsc.VectorSubcoreMesh(core_axis_name="c", subcore_axis_name="s")
    return pl.kernel(
        _body,
        mesh=mesh,
        out_type=jax.ShapeDtypeStruct((B * S * D,), jnp.float32),
        scratch_types=[
            pltpu.VMEM((CW,), jnp.float32),
            pltpu.VMEM((CW,), jnp.float32),
        ],
    )(x_flat, pos_flat)


def kernel(x, pos_table):
    out = _pos_add(x.reshape(-1), pos_table.reshape(-1))
    return out.reshape(B, S, D)


# speedup vs baseline: 1.1918x; 1.1918x over previous
import jax
import jax.numpy as jnp
from jax import lax
from jax.experimental import pallas as pl
from jax.experimental.pallas import tpu as pltpu
from jax.experimental.pallas import tpu_sc as plsc

B, S, D = 4, 8192, 768
NC, NS = 2, 16
NW = NC * NS
S_PER_W = S // NW
CHUNK = 32
STEPS = S_PER_W // CHUNK
CW = CHUNK * D
LANES = 16
UNROLL = 8
K = STEPS * B
NXB = 3


def _body(x_hbm, pos_hbm, out_hbm,
          x0, x1, x2, p0, p1,
          xin0, xin1, xin2, xout0, xout1, xout2, ps0, ps1):
    xb = [x0, x1, x2]
    pb = [p0, p1]
    xin = [xin0, xin1, xin2]
    xout = [xout0, xout1, xout2]
    ps = [ps0, ps1]

    wid = lax.axis_index("s") * NC + lax.axis_index("c")
    base = wid * S_PER_W * D

    def p_off(t):
        return base + t * CW

    def x_off(k):
        return (k % B) * (S * D) + p_off(k // B)

    pending_in = {}
    pending_out = {}
    pending_p = {}

    def start_p(t):
        pending_p[t] = pltpu.async_copy(
            pos_hbm.at[pl.ds(p_off(t), CW)], pb[t % 2], ps[t % 2])

    def start_in(k):
        pending_in[k] = pltpu.async_copy(
            x_hbm.at[pl.ds(x_off(k), CW)], xb[k % NXB], xin[k % NXB])

    def start_out(k):
        pending_out[k] = pltpu.async_copy(
            xb[k % NXB], out_hbm.at[pl.ds(x_off(k), CW)], xout[k % NXB])

    start_p(0)
    start_in(0)
    start_in(1)

    for k in range(K):
        t, b = k // B, k % B
        if b == 0:
            pending_p.pop(t).wait()
            if t + 1 < STEPS:
                start_p(t + 1)
        pending_in.pop(k).wait()

        xv, pv = xb[k % NXB], pb[t % 2]

        def add_body(i, _, xv=xv, pv=pv):
            o0 = i * (LANES * UNROLL)
            for u in range(UNROLL):
                o = o0 + u * LANES
                xv[pl.ds(o, LANES)] = xv[pl.ds(o, LANES)] + pv[pl.ds(o, LANES)]
            return 0

        lax.fori_loop(0, CW // (LANES * UNROLL), add_body, 0)

        start_out(k)
        if k + 2 < K:
            if k - 1 >= 0:
                pending_out.pop(k - 1).wait()
            start_in(k + 2)

    for k in sorted(pending_out):
        pending_out.pop(k).wait()


@jax.jit
def _pos_add(x_flat, pos_flat):
    mesh = plsc.VectorSubcoreMesh(core_axis_name="c", subcore_axis_name="s")
    return pl.kernel(
        _body,
        mesh=mesh,
        out_type=jax.ShapeDtypeStruct((B * S * D,), jnp.float32),
        scratch_types=(
            [pltpu.VMEM((CW,), jnp.float32)] * (NXB + 2)
            + [pltpu.SemaphoreType.DMA] * (2 * NXB + 2)
        ),
    )(x_flat, pos_flat)


def kernel(x, pos_table):
    out = _pos_add(x.reshape(-1), pos_table.reshape(-1))
    return out.reshape(B, S, D)
